# baseline (speedup 1.0000x reference)
import functools

import jax
import jax.numpy as jnp
from jax import lax
from jax.experimental import pallas as pl
from jax.experimental.pallas import tpu as pltpu
from jax.experimental.pallas import tpu_sc as plsc

B = 1024
FEAT = 3 * 64 * 64
RB = 512


@functools.lru_cache(maxsize=1)
def _make_sc_gather():
    info = plsc.get_sparse_core_info()
    nc, ns = info.num_cores, info.num_subcores
    nw = nc * ns
    bpw = B // nw

    mesh = plsc.VectorSubcoreMesh(core_axis_name="c", subcore_axis_name="s")

    @functools.partial(
        pl.kernel,
        mesh=mesh,
        out_type=[
            jax.ShapeDtypeStruct((B,), jnp.float32),
            jax.ShapeDtypeStruct((B,), jnp.float32),
        ],
        scratch_types=[
            pltpu.VMEM((bpw,), jnp.int32),
            pltpu.VMEM((bpw,), jnp.float32),
            pltpu.VMEM((bpw,), jnp.float32),
            pltpu.SemaphoreType.DMA,
        ],
    )
    def sc_gather(t_hbm, sa_hbm, sb_hbm, out_a_hbm, out_b_hbm, idx_v, a_v, b_v, sem):
        wid = lax.axis_index("s") * nc + lax.axis_index("c")
        base = wid * bpw
        pltpu.sync_copy(t_hbm.at[pl.ds(base, bpw)], idx_v)
        ca = pltpu.async_copy(sa_hbm.at[idx_v], a_v, sem)
        cb = pltpu.async_copy(sb_hbm.at[idx_v], b_v, sem)
        ca.wait()
        cb.wait()
        pltpu.sync_copy(a_v, out_a_hbm.at[pl.ds(base, bpw)])
        pltpu.sync_copy(b_v, out_b_hbm.at[pl.ds(base, bpw)])

    return sc_gather


def _tc_body(sa_ref, sb_ref, xa_ref, xb_ref, xc_ref, ea_ref, eb_ref, ec_ref, out_ref):
    out_ref[:RB, :] = sa_ref[...] * xa_ref[...] + sb_ref[...] * ea_ref[...]
    out_ref[RB:2 * RB, :] = sa_ref[...] * xb_ref[...] + sb_ref[...] * eb_ref[...]
    out_ref[2 * RB:, :] = sa_ref[...] * xc_ref[...] + sb_ref[...] * ec_ref[...]


def _tc_apply(sa_g, sb_g, xt, et):
    grid = (FEAT // (3 * RB),)
    coef_spec = pl.BlockSpec((1, B), lambda i: (0, 0))
    in_a = pl.BlockSpec((RB, B), lambda i: (3 * i, 0))
    in_b = pl.BlockSpec((RB, B), lambda i: (3 * i + 1, 0))
    in_c = pl.BlockSpec((RB, B), lambda i: (3 * i + 2, 0))
    out_spec = pl.BlockSpec((3 * RB, B), lambda i: (i, 0))
    return pl.pallas_call(
        _tc_body,
        grid=grid,
        in_specs=[coef_spec, coef_spec, in_a, in_b, in_c, in_a, in_b, in_c],
        out_specs=out_spec,
        out_shape=jax.ShapeDtypeStruct((FEAT, B), jnp.float32),
    )(sa_g.reshape(1, B), sb_g.reshape(1, B), xt, xt, xt, et, et, et)


def kernel(x0, t, eps, sqrt_alpha_bar, sqrt_one_minus_alpha_bar):
    c, h, w = x0.shape[1:]
    xt = x0.transpose(1, 2, 3, 0).reshape(FEAT, B)
    et = eps.transpose(1, 2, 3, 0).reshape(FEAT, B)
    sa_g, sb_g = _make_sc_gather()(t.astype(jnp.int32), sqrt_alpha_bar,
                                   sqrt_one_minus_alpha_bar)
    out = _tc_apply(sa_g, sb_g, xt, et)
    return out.reshape(c, h, w, B).transpose(3, 0, 1, 2)


# speedup vs baseline: 1.0048x; 1.0048x over previous
import functools

import jax
import jax.numpy as jnp
from jax import lax
from jax.experimental import pallas as pl
from jax.experimental.pallas import tpu as pltpu
from jax.experimental.pallas import tpu_sc as plsc

B = 1024
FEAT = 3 * 64 * 64
RB = 512


@functools.lru_cache(maxsize=1)
def _make_sc_gather():
    info = plsc.get_sparse_core_info()
    nc, ns = info.num_cores, info.num_subcores
    nw = nc * ns
    bpw = B // nw

    mesh = plsc.VectorSubcoreMesh(core_axis_name="c", subcore_axis_name="s")

    @functools.partial(
        pl.kernel,
        mesh=mesh,
        out_type=[
            jax.ShapeDtypeStruct((B,), jnp.float32),
            jax.ShapeDtypeStruct((B,), jnp.float32),
        ],
        scratch_types=[
            pltpu.VMEM((bpw,), jnp.int32),
            pltpu.VMEM((bpw,), jnp.float32),
            pltpu.VMEM((bpw,), jnp.float32),
            pltpu.SemaphoreType.DMA,
        ],
    )
    def sc_gather(t_hbm, sa_hbm, sb_hbm, out_a_hbm, out_b_hbm, idx_v, a_v, b_v, sem):
        wid = lax.axis_index("s") * nc + lax.axis_index("c")
        base = wid * bpw
        pltpu.sync_copy(t_hbm.at[pl.ds(base, bpw)], idx_v)
        ca = pltpu.async_copy(sa_hbm.at[idx_v], a_v, sem)
        cb = pltpu.async_copy(sb_hbm.at[idx_v], b_v, sem)
        ca.wait()
        cb.wait()
        pltpu.sync_copy(a_v, out_a_hbm.at[pl.ds(base, bpw)])
        pltpu.sync_copy(b_v, out_b_hbm.at[pl.ds(base, bpw)])

    return sc_gather


def _tc_body(sa_ref, sb_ref, xa_ref, xb_ref, ea_ref, eb_ref, out_ref):
    out_ref[:RB, :] = sa_ref[...] * xa_ref[...] + sb_ref[...] * ea_ref[...]
    out_ref[RB:, :] = sa_ref[...] * xb_ref[...] + sb_ref[...] * eb_ref[...]


def _tc_apply(sa_g, sb_g, xt, et):
    grid = (FEAT // (2 * RB),)
    coef_spec = pl.BlockSpec((1, B), lambda i: (0, 0))
    in_a = pl.BlockSpec((RB, B), lambda i: (2 * i, 0))
    in_b = pl.BlockSpec((RB, B), lambda i: (2 * i + 1, 0))
    out_spec = pl.BlockSpec((2 * RB, B), lambda i: (i, 0))
    return pl.pallas_call(
        _tc_body,
        grid=grid,
        in_specs=[coef_spec, coef_spec, in_a, in_b, in_a, in_b],
        out_specs=out_spec,
        out_shape=jax.ShapeDtypeStruct((FEAT, B), jnp.float32),
    )(sa_g.reshape(1, B), sb_g.reshape(1, B), xt, xt, et, et)


def kernel(x0, t, eps, sqrt_alpha_bar, sqrt_one_minus_alpha_bar):
    c, h, w = x0.shape[1:]
    xt = x0.transpose(1, 2, 3, 0).reshape(FEAT, B)
    et = eps.transpose(1, 2, 3, 0).reshape(FEAT, B)
    sa_g, sb_g = _make_sc_gather()(t.astype(jnp.int32), sqrt_alpha_bar,
                                   sqrt_one_minus_alpha_bar)
    out = _tc_apply(sa_g, sb_g, xt, et)
    return out.reshape(c, h, w, B).transpose(3, 0, 1, 2)
